# baseline (speedup 1.0000x reference)
import functools

import jax
import jax.numpy as jnp
from jax import lax
from jax.experimental import pallas as pl
from jax.experimental.pallas import tpu as pltpu
from jax.experimental.pallas import tpu_sc as plsc

NUM_ROWS = 1_000_000
D = 64
NB = 16384
NJ = 26
NW = 32
CHUNK = 128
NJ_HALF = NJ // 2
HALF_ROWS = NB * NJ_HALF
B_PER_W = HALF_ROWS // NW
N_CHUNKS = B_PER_W // CHUNK
NBUF = 4
N_GROUPS = N_CHUNKS // NBUF


def _emb_kernel(idx_hbm, table_hbm, out_hbm, idx_v, rows, gsems, ssems,
                idx_sem):
    wid = lax.axis_index("s") * 2 + lax.axis_index("c")
    base = wid * B_PER_W

    pltpu.async_copy(idx_hbm.at[pl.ds(base, B_PER_W)], idx_v, idx_sem).wait()

    def gather_refs(j, b):
        return (table_hbm.at[idx_v.at[pl.ds(j * CHUNK, CHUNK)]], rows.at[b],
                gsems.at[b])

    def store_refs(j, b):
        return (rows.at[b], out_hbm.at[pl.ds(base + j * CHUNK, CHUNK)],
                ssems.at[b])

    for b in range(NBUF):
        pltpu.async_copy(*gather_refs(b, b))

    def group(g, carry):
        for b in range(NBUF):
            j = g * NBUF + b
            pltpu.make_async_copy(*gather_refs(j, b)).wait()
            pltpu.async_copy(*store_refs(j, b))

            @pl.when(g < N_GROUPS - 1)
            def _():
                pltpu.make_async_copy(*store_refs(j, b)).wait()
                pltpu.async_copy(*gather_refs(j + NBUF, b))
        return carry

    lax.fori_loop(0, N_GROUPS, group, 0)

    for b in range(NBUF):
        j = (N_GROUPS - 1) * NBUF + b
        pltpu.make_async_copy(*store_refs(j, b)).wait()


def _lookup_half(idx, weight):
    mesh = plsc.VectorSubcoreMesh(core_axis_name="c", subcore_axis_name="s")
    return pl.kernel(
        _emb_kernel,
        out_type=jax.ShapeDtypeStruct((HALF_ROWS, D), jnp.float32),
        mesh=mesh,
        scratch_types=[
            pltpu.VMEM((B_PER_W,), jnp.int32),
            pltpu.VMEM((NBUF, CHUNK, D), jnp.float32),
            pltpu.SemaphoreType.DMA((NBUF,)),
            pltpu.SemaphoreType.DMA((NBUF,)),
            pltpu.SemaphoreType.DMA,
        ],
        compiler_params=pltpu.CompilerParams(
            use_tc_tiling_on_sc=False, needs_layout_passes=False),
    )(idx, weight)


@jax.jit
def kernel(Z, weight):
    Zi = Z.astype(jnp.int32)
    halves = []
    for h in range(2):
        idx = Zi[:, h * NJ_HALF:(h + 1) * NJ_HALF].reshape(HALF_ROWS)
        out = _lookup_half(idx, weight)
        halves.append(out.reshape(NB, NJ_HALF, D))
    return jnp.concatenate(halves, axis=1)


# speedup vs baseline: 1.0495x; 1.0495x over previous
import jax
import jax.numpy as jnp
from jax import lax
from jax.experimental import pallas as pl
from jax.experimental.pallas import tpu as pltpu
from jax.experimental.pallas import tpu_sc as plsc

NUM_ROWS = 1_000_000
D = 64
NB = 16384
NJ = 26
B_TOTAL = NB * NJ
NW = 32
B_PER_W = B_TOTAL // NW
CHUNK = 128
N_CHUNKS = B_PER_W // CHUNK
NBUF = 8
N_GROUPS = N_CHUNKS // NBUF


def _emb_kernel(idx_hbm, table_hbm, out_hbm, idx_v, rows, gsems, ssems,
                idx_sem):
    wid = lax.axis_index("s") * 2 + lax.axis_index("c")
    base = wid * B_PER_W

    pltpu.async_copy(idx_hbm.at[pl.ds(base, B_PER_W)], idx_v, idx_sem).wait()

    def gather_refs(j, b):
        return (table_hbm.at[idx_v.at[pl.ds(j * CHUNK, CHUNK)]], rows.at[b],
                gsems.at[b])

    def store_refs(j, b):
        return (rows.at[b], out_hbm.at[pl.ds(base + j * CHUNK, CHUNK)],
                ssems.at[b])

    for b in range(NBUF):
        pltpu.async_copy(*gather_refs(b, b))

    def group(g, carry):
        for b in range(NBUF):
            j = g * NBUF + b
            pltpu.make_async_copy(*gather_refs(j, b)).wait()
            pltpu.async_copy(*store_refs(j, b))

            @pl.when(g < N_GROUPS - 1)
            def _():
                pltpu.make_async_copy(*store_refs(j, b)).wait()
                pltpu.async_copy(*gather_refs(j + NBUF, b))
        return carry

    lax.fori_loop(0, N_GROUPS, group, 0)

    for b in range(NBUF):
        j = (N_GROUPS - 1) * NBUF + b
        pltpu.make_async_copy(*store_refs(j, b)).wait()


@jax.jit
def kernel(Z, weight):
    idx = Z.astype(jnp.int32).reshape(B_TOTAL)
    mesh = plsc.VectorSubcoreMesh(core_axis_name="c", subcore_axis_name="s")
    out = pl.kernel(
        _emb_kernel,
        out_type=jax.ShapeDtypeStruct((B_TOTAL, D), jnp.float32),
        mesh=mesh,
        scratch_types=[
            pltpu.VMEM((B_PER_W,), jnp.int32),
            pltpu.VMEM((NBUF, CHUNK, D), jnp.float32),
            pltpu.SemaphoreType.DMA((NBUF,)),
            pltpu.SemaphoreType.DMA((NBUF,)),
            pltpu.SemaphoreType.DMA,
        ],
        compiler_params=pltpu.CompilerParams(
            use_tc_tiling_on_sc=False, needs_layout_passes=False),
    )(idx, weight)
    return out.reshape(NB, NJ, D)
